# baseline (speedup 1.0000x reference)
import functools

import jax
import jax.numpy as jnp
from jax import lax
from jax.experimental import pallas as pl
from jax.experimental.pallas import tpu as pltpu
from jax.experimental.pallas import tpu_sc as plsc

B = 16384
D = 32
TR = 8
NC, NS, L = 2, 16, 16
NW = NC * NS
BPW = B // NW
C = L
NCHUNK = BPW // C

_mesh = plsc.VectorSubcoreMesh(
    core_axis_name="c", subcore_axis_name="s", num_cores=NC, num_subcores=NS
)


@functools.partial(
    pl.kernel,
    out_type=jax.ShapeDtypeStruct((B,), jnp.float32),
    mesh=_mesh,
    scratch_types=[
        pltpu.VMEM((BPW,), jnp.int32),
        pltpu.VMEM((BPW,), jnp.int32),
        pltpu.VMEM((BPW,), jnp.int32),
        pltpu.VMEM((BPW,), jnp.int32),
        pltpu.VMEM((BPW,), jnp.int32),
        pltpu.VMEM((C, TR, D), jnp.float32),
        pltpu.VMEM((C, TR, D), jnp.float32),
        pltpu.VMEM((C, TR, D), jnp.float32),
        pltpu.VMEM((C, TR, D), jnp.float32),
        pltpu.VMEM((C, TR, D), jnp.float32),
        pltpu.VMEM((BPW,), jnp.float32),
        pltpu.VMEM((BPW,), jnp.float32),
        pltpu.VMEM((BPW,), jnp.float32),
        pltpu.SemaphoreType.DMA,
    ],
    compiler_params=pltpu.CompilerParams(
        needs_layout_passes=False, disable_bounds_checks=True
    ),
)
def _sc_score(
    user_hbm, item_hbm, auth_hbm, genr_hbm, subj_hbm,
    ub_hbm, ib_hbm, ue_hbm, ie_hbm, ae_hbm, ge_hbm, se_hbm,
    out_hbm,
    idx_u, idx_i, idx_a, idx_g, idx_s,
    rows_u, rows_i, rows_a, rows_g, rows_s,
    bias_u, bias_i, out_v, sem,
):
    wid = lax.axis_index("s") * NC + lax.axis_index("c")
    base = wid * BPW

    vue = ue_hbm.reshape(ue_hbm.shape[0] // TR, TR, D)
    vie = ie_hbm.reshape(ie_hbm.shape[0] // TR, TR, D)
    vae = ae_hbm.reshape(ae_hbm.shape[0] // TR, TR, D)
    vge = ge_hbm.reshape(ge_hbm.shape[0] // TR, TR, D)
    vse = se_hbm.reshape(se_hbm.shape[0] // TR, TR, D)

    pltpu.sync_copy(user_hbm.at[pl.ds(base, BPW)], idx_u)
    pltpu.sync_copy(item_hbm.at[pl.ds(base, BPW)], idx_i)
    pltpu.sync_copy(auth_hbm.at[pl.ds(base, BPW)], idx_a)
    pltpu.sync_copy(genr_hbm.at[pl.ds(base, BPW)], idx_g)
    pltpu.sync_copy(subj_hbm.at[pl.ds(base, BPW)], idx_s)

    bu_cp = pltpu.async_copy(ub_hbm.at[idx_u], bias_u, sem)
    bi_cp = pltpu.async_copy(ib_hbm.at[idx_i], bias_i, sem)
    bu_cp.wait()
    bi_cp.wait()

    lane = lax.iota(jnp.int32, L)

    def chunk(k, carry):
        sl = pl.ds(k * C, C)
        viu = idx_u[sl]
        vii = idx_i[sl]
        via = idx_a[sl]
        vig = idx_g[sl]
        vis = idx_s[sl]
        vqu = viu >> 3
        vqi = vii >> 3
        vqa = via >> 3
        vqg = vig >> 3
        vqs = vis >> 3
        for j in range(C):
            pltpu.async_copy(vue.at[vqu[j]], rows_u.at[j], sem)
            pltpu.async_copy(vie.at[vqi[j]], rows_i.at[j], sem)
            pltpu.async_copy(vae.at[vqa[j]], rows_a.at[j], sem)
            pltpu.async_copy(vge.at[vqg[j]], rows_g.at[j], sem)
            pltpu.async_copy(vse.at[vqs[j]], rows_s.at[j], sem)
        pltpu.make_async_copy(vue.at[pl.ds(0, C)], rows_u, sem).wait()
        pltpu.make_async_copy(vie.at[pl.ds(0, C)], rows_i, sem).wait()
        pltpu.make_async_copy(vae.at[pl.ds(0, C)], rows_a, sem).wait()
        pltpu.make_async_copy(vge.at[pl.ds(0, C)], rows_g, sem).wait()
        pltpu.make_async_copy(vse.at[pl.ds(0, C)], rows_s, sem).wait()

        ru = viu & (TR - 1)
        ri = vii & (TR - 1)
        ra = via & (TR - 1)
        rg = vig & (TR - 1)
        rs = vis & (TR - 1)
        acc = bias_u[sl] + bias_i[sl]
        for f in range(D):
            fv = jnp.full((L,), f, jnp.int32)
            cu = plsc.load_gather(rows_u, [lane, ru, fv])
            ci = plsc.load_gather(rows_i, [lane, ri, fv])
            ca = plsc.load_gather(rows_a, [lane, ra, fv])
            cg = plsc.load_gather(rows_g, [lane, rg, fv])
            cs = plsc.load_gather(rows_s, [lane, rs, fv])
            acc = acc + cu * (ci + ca + cg + cs)
        out_v[sl] = acc
        return carry

    lax.fori_loop(0, NCHUNK, chunk, 0)

    pltpu.sync_copy(out_v, out_hbm.at[pl.ds(base, BPW)])


def kernel(user, item, item_authors_in, item_authors_off, item_genres_in,
           item_genres_off, item_subjects_in, item_subjects_off,
           u_bias_w, i_bias_w, u_embed_w, i_embed_w, a_embed_w, g_embed_w,
           s_embed_w):
    del item_authors_off, item_genres_off, item_subjects_off
    return _sc_score(
        user.astype(jnp.int32),
        item.astype(jnp.int32),
        item_authors_in.astype(jnp.int32),
        item_genres_in.astype(jnp.int32),
        item_subjects_in.astype(jnp.int32),
        u_bias_w.T.reshape(-1),
        i_bias_w.T.reshape(-1),
        u_embed_w, i_embed_w, a_embed_w, g_embed_w, s_embed_w,
    )


# speedup vs baseline: 1.0030x; 1.0030x over previous
import functools

import jax
import jax.numpy as jnp
from jax import lax
from jax.experimental import pallas as pl
from jax.experimental.pallas import tpu as pltpu
from jax.experimental.pallas import tpu_sc as plsc

B = 16384
D = 32
TR = 8
NC, NS, L = 2, 16, 16
NW = NC * NS
BPW = B // NW
C = L
NCHUNK = BPW // C

_mesh = plsc.VectorSubcoreMesh(
    core_axis_name="c", subcore_axis_name="s", num_cores=NC, num_subcores=NS
)


@functools.partial(
    pl.kernel,
    out_type=jax.ShapeDtypeStruct((B,), jnp.float32),
    mesh=_mesh,
    scratch_types=[
        pltpu.VMEM((BPW,), jnp.int32),
        pltpu.VMEM((BPW,), jnp.int32),
        pltpu.VMEM((BPW,), jnp.int32),
        pltpu.VMEM((BPW,), jnp.int32),
        pltpu.VMEM((BPW,), jnp.int32),
        pltpu.VMEM((C, TR, D), jnp.float32),
        pltpu.VMEM((C, TR, D), jnp.float32),
        pltpu.VMEM((C, TR, D), jnp.float32),
        pltpu.VMEM((C, TR, D), jnp.float32),
        pltpu.VMEM((C, TR, D), jnp.float32),
        pltpu.VMEM((BPW,), jnp.float32),
        pltpu.VMEM((BPW,), jnp.float32),
        pltpu.VMEM((BPW,), jnp.float32),
        pltpu.SemaphoreType.DMA,
    ],
    compiler_params=pltpu.CompilerParams(
        needs_layout_passes=False, disable_bounds_checks=True
    ),
)
def _sc_score(
    user_hbm, item_hbm, auth_hbm, genr_hbm, subj_hbm,
    ub_hbm, ib_hbm, ue_hbm, ie_hbm, ae_hbm, ge_hbm, se_hbm,
    out_hbm,
    idx_u, idx_i, idx_a, idx_g, idx_s,
    rows_u, rows_i, rows_a, rows_g, rows_s,
    bias_u, bias_i, out_v, sem,
):
    wid = lax.axis_index("s") * NC + lax.axis_index("c")
    base = wid * BPW

    vue = ue_hbm.reshape(ue_hbm.shape[0] // TR, TR, D)
    vie = ie_hbm.reshape(ie_hbm.shape[0] // TR, TR, D)
    vae = ae_hbm.reshape(ae_hbm.shape[0] // TR, TR, D)
    vge = ge_hbm.reshape(ge_hbm.shape[0] // TR, TR, D)
    vse = se_hbm.reshape(se_hbm.shape[0] // TR, TR, D)

    pltpu.sync_copy(user_hbm.at[pl.ds(base, BPW)], idx_u)
    pltpu.sync_copy(item_hbm.at[pl.ds(base, BPW)], idx_i)
    pltpu.sync_copy(auth_hbm.at[pl.ds(base, BPW)], idx_a)
    pltpu.sync_copy(genr_hbm.at[pl.ds(base, BPW)], idx_g)
    pltpu.sync_copy(subj_hbm.at[pl.ds(base, BPW)], idx_s)

    bu_cp = pltpu.async_copy(ub_hbm.at[idx_u], bias_u, sem)
    bi_cp = pltpu.async_copy(ib_hbm.at[idx_i], bias_i, sem)
    bu_cp.wait()
    bi_cp.wait()

    lane = lax.iota(jnp.int32, L)

    def chunk(k, carry):
        sl = pl.ds(k * C, C)
        viu = idx_u[sl]
        vii = idx_i[sl]
        via = idx_a[sl]
        vig = idx_g[sl]
        vis = idx_s[sl]
        vqu = viu >> 3
        vqi = vii >> 3
        vqa = via >> 3
        vqg = vig >> 3
        vqs = vis >> 3
        for j in range(C):
            pltpu.async_copy(vue.at[vqu[j]], rows_u.at[j], sem)
            pltpu.async_copy(vie.at[vqi[j]], rows_i.at[j], sem)
            pltpu.async_copy(vae.at[vqa[j]], rows_a.at[j], sem)
            pltpu.async_copy(vge.at[vqg[j]], rows_g.at[j], sem)
            pltpu.async_copy(vse.at[vqs[j]], rows_s.at[j], sem)
        pltpu.make_async_copy(vue.at[pl.ds(0, C)], rows_u, sem).wait()
        pltpu.make_async_copy(vie.at[pl.ds(0, C)], rows_i, sem).wait()
        pltpu.make_async_copy(vae.at[pl.ds(0, C)], rows_a, sem).wait()
        pltpu.make_async_copy(vge.at[pl.ds(0, C)], rows_g, sem).wait()
        pltpu.make_async_copy(vse.at[pl.ds(0, C)], rows_s, sem).wait()

        ru = viu & (TR - 1)
        ri = vii & (TR - 1)
        ra = via & (TR - 1)
        rg = vig & (TR - 1)
        rs = vis & (TR - 1)
        acc = bias_u[sl] + bias_i[sl]
        for f in range(D):
            fv = jnp.full((L,), f, jnp.int32)
            cu = plsc.load_gather(rows_u, [lane, ru, fv])
            ci = plsc.load_gather(rows_i, [lane, ri, fv])
            ca = plsc.load_gather(rows_a, [lane, ra, fv])
            cg = plsc.load_gather(rows_g, [lane, rg, fv])
            cs = plsc.load_gather(rows_s, [lane, rs, fv])
            acc = acc + cu * (ci + ca + cg + cs)
        out_v[sl] = acc
        return carry

    lax.fori_loop(0, NCHUNK, chunk, 0)

    pltpu.sync_copy(out_v, out_hbm.at[pl.ds(base, BPW)])


def kernel(user, item, item_authors_in, item_authors_off, item_genres_in,
           item_genres_off, item_subjects_in, item_subjects_off,
           u_bias_w, i_bias_w, u_embed_w, i_embed_w, a_embed_w, g_embed_w,
           s_embed_w):
    del item_authors_off, item_genres_off, item_subjects_off
    return _sc_score(
        user.astype(jnp.int32),
        item.astype(jnp.int32),
        item_authors_in.astype(jnp.int32),
        item_genres_in.astype(jnp.int32),
        item_subjects_in.astype(jnp.int32),
        lax.squeeze(u_bias_w, (1,)),
        lax.squeeze(i_bias_w, (1,)),
        u_embed_w, i_embed_w, a_embed_w, g_embed_w, s_embed_w,
    )
